# baseline (speedup 1.0000x reference)
import functools

import jax
import jax.numpy as jnp
from jax import lax
from jax.experimental import pallas as pl
from jax.experimental.pallas import tpu as pltpu
from jax.experimental.pallas import tpu_sc as plsc

D = 128
B = 16384
T_ROWS = 128

NC = 2
NS = 16
NW = NC * NS
ROWS_PER_W = B // NW
K = 64
N_CHUNKS = ROWS_PER_W // K
LANES = 16
DCH = D // LANES

_GDN = jax.lax.GatherDimensionNumbers(
    offset_dims=(), collapsed_slice_dims=(0,), start_index_map=(0,))


def _permute(x, idx):
    return lax.gather(x, idx[:, None], _GDN, (1,),
                      mode=lax.GatherScatterMode.PROMISE_IN_BOUNDS)


def _bcast_sum(x):
    lane = lax.iota(jnp.int32, LANES)
    for d in (8, 4, 2, 1):
        x = x + _permute(x, lane ^ d)
    return x


def _sc_kernel(ent_hbm, rel_hbm, time_hbm, idx_hbm,
               pos_hbm, neg_hbm,
               ph_rows, pt_rows, rl_rows, nh_rows, nt_rows,
               time_vmem, idx_buf,
               pos_buf, neg_buf,
               s0, s1, s2, s3, s4):
    wid = lax.axis_index("s") * NC + lax.axis_index("c")
    base = wid * ROWS_PER_W

    tables = (ent_hbm, ent_hbm, rel_hbm, ent_hbm, ent_hbm)
    row_bufs = (ph_rows, pt_rows, rl_rows, nh_rows, nt_rows)
    sems = (s0, s1, s2, s3, s4)

    pltpu.sync_copy(time_hbm, time_vmem)
    pltpu.sync_copy(idx_hbm.at[:, pl.ds(base, ROWS_PER_W)], idx_buf)

    def fire(g, slot):
        for a, (tbl, rb, sem) in enumerate(zip(tables, row_bufs, sems)):
            pltpu.async_copy(
                tbl.at[idx_buf.at[a, pl.ds(g * K, K)]], rb.at[slot], sem)

    def drain(g, slot):
        for a, (tbl, rb, sem) in enumerate(zip(tables, row_bufs, sems)):
            pltpu.make_async_copy(
                tbl.at[idx_buf.at[a, pl.ds(g * K, K)]], rb.at[slot], sem).wait()

    fire(0, 0)
    lane = lax.iota(jnp.int32, LANES)

    def chunk_body(g, carry):
        slot = lax.rem(g, 2)
        drain(g, slot)

        @pl.when(g + 1 < N_CHUNKS)
        def _prefetch():
            fire(g + 1, lax.rem(g + 1, 2))

        def grp_body(grp, c2):
            rbase = g * K + grp * LANES
            lbase = grp * LANES
            pos_vec = jnp.zeros((LANES,), jnp.float32)
            neg_vec = jnp.zeros((LANES,), jnp.float32)
            sy_vec = idx_buf[5, pl.ds(rbase, LANES)]
            for i in range(LANES):
                r = lbase + i
                sy = sy_vec[i]
                ts, us, vs = [], [], []
                du = jnp.zeros((LANES,), jnp.float32)
                dv = jnp.zeros((LANES,), jnp.float32)
                for j in range(DCH):
                    sl = pl.ds(j * LANES, LANES)
                    t = time_vmem[sy, sl]
                    rr = rl_rows[slot, r, sl]
                    u = ph_rows[slot, r, sl] + rr - pt_rows[slot, r, sl]
                    v = nh_rows[slot, r, sl] + rr - nt_rows[slot, r, sl]
                    du = du + u * t
                    dv = dv + v * t
                    ts.append(t)
                    us.append(u)
                    vs.append(v)
                du = _bcast_sum(du)
                dv = _bcast_sum(dv)
                pa = jnp.zeros((LANES,), jnp.float32)
                na = jnp.zeros((LANES,), jnp.float32)
                for j in range(DCH):
                    pa = pa + jnp.abs(us[j] - ts[j] * du)
                    na = na + jnp.abs(vs[j] - ts[j] * dv)
                pos_vec = jnp.where(lane == i, _bcast_sum(pa), pos_vec)
                neg_vec = jnp.where(lane == i, _bcast_sum(na), neg_vec)
            pos_buf[pl.ds(rbase, LANES)] = pos_vec
            neg_buf[pl.ds(rbase, LANES)] = neg_vec
            return c2

        lax.fori_loop(0, K // LANES, grp_body, 0)
        return carry

    lax.fori_loop(0, N_CHUNKS, chunk_body, 0)

    pltpu.sync_copy(pos_buf, pos_hbm.at[pl.ds(base, ROWS_PER_W)])
    pltpu.sync_copy(neg_buf, neg_hbm.at[pl.ds(base, ROWS_PER_W)])


@jax.jit
def kernel(ent_embeddings, rel_embeddings, time_embeddings,
           pos_head, pos_tail, rel, neg_head, neg_tail, start_year):
    mesh = plsc.VectorSubcoreMesh(core_axis_name="c", subcore_axis_name="s")
    fn = functools.partial(
        pl.kernel, mesh=mesh,
        out_type=(jax.ShapeDtypeStruct((B,), jnp.float32),
                  jax.ShapeDtypeStruct((B,), jnp.float32)),
        scratch_types=(
            [pltpu.VMEM((2, K, D), jnp.float32)] * 5
            + [pltpu.VMEM((T_ROWS, D), jnp.float32)]
            + [pltpu.VMEM((6, ROWS_PER_W), jnp.int32)]
            + [pltpu.VMEM((ROWS_PER_W,), jnp.float32)] * 2
            + [pltpu.SemaphoreType.DMA] * 5
        ),
    )(_sc_kernel)
    idx_all = jnp.stack([pos_head[:, 0], pos_tail[:, 0], rel[:, 0],
                         neg_head[:, 0], neg_tail[:, 0], start_year])
    pos, neg = fn(ent_embeddings, rel_embeddings, time_embeddings, idx_all)
    return pos.reshape(B, 1), neg.reshape(B, 1)


# speedup vs baseline: 1.0121x; 1.0121x over previous
import functools

import jax
import jax.numpy as jnp
from jax import lax
from jax.experimental import pallas as pl
from jax.experimental.pallas import tpu as pltpu
from jax.experimental.pallas import tpu_sc as plsc

D = 128
B = 16384
T_ROWS = 128

NC = 2
NS = 16
NW = NC * NS
ROWS_PER_W = B // NW
K = 64
N_CHUNKS = ROWS_PER_W // K
LANES = 16
DCH = D // LANES

_GDN = jax.lax.GatherDimensionNumbers(
    offset_dims=(), collapsed_slice_dims=(0,), start_index_map=(0,))


def _permute(x, idx):
    return lax.gather(x, idx[:, None], _GDN, (1,),
                      mode=lax.GatherScatterMode.PROMISE_IN_BOUNDS)


def _bcast_sum(x):
    lane = lax.iota(jnp.int32, LANES)
    for d in (8, 4, 2, 1):
        x = x + _permute(x, lane ^ d)
    return x


def _sc_kernel(ent_hbm, rel_hbm, time_hbm, idx_hbm,
               pos_hbm, neg_hbm,
               ph_rows, pt_rows, rl_rows, nh_rows, nt_rows,
               time_vmem, idx_buf, rel_sp,
               pos_buf, neg_buf,
               s0, s1, s2, s3, s4):
    sid = lax.axis_index("s")
    wid = sid * NC + lax.axis_index("c")
    base = wid * ROWS_PER_W

    tables = (ent_hbm, ent_hbm, rel_sp, ent_hbm, ent_hbm)
    row_bufs = (ph_rows, pt_rows, rl_rows, nh_rows, nt_rows)
    sems = (s0, s1, s2, s3, s4)

    @pl.when(sid == 0)
    def _stage_rel():
        pltpu.sync_copy(rel_hbm, rel_sp)

    pltpu.sync_copy(time_hbm, time_vmem)
    pltpu.sync_copy(idx_hbm.at[:, pl.ds(base, ROWS_PER_W)], idx_buf)
    plsc.subcore_barrier()

    def fire(g, slot):
        for a, (tbl, rb, sem) in enumerate(zip(tables, row_bufs, sems)):
            pltpu.async_copy(
                tbl.at[idx_buf.at[a, pl.ds(g * K, K)]], rb.at[slot], sem)

    def drain(g, slot):
        for a, (tbl, rb, sem) in enumerate(zip(tables, row_bufs, sems)):
            pltpu.make_async_copy(
                tbl.at[idx_buf.at[a, pl.ds(g * K, K)]], rb.at[slot], sem).wait()

    fire(0, 0)
    lane = lax.iota(jnp.int32, LANES)

    def chunk_body(g, carry):
        slot = lax.rem(g, 2)
        drain(g, slot)

        @pl.when(g + 1 < N_CHUNKS)
        def _prefetch():
            fire(g + 1, lax.rem(g + 1, 2))

        def grp_body(grp, c2):
            rbase = g * K + grp * LANES
            lbase = grp * LANES
            pos_vec = jnp.zeros((LANES,), jnp.float32)
            neg_vec = jnp.zeros((LANES,), jnp.float32)
            sy_vec = idx_buf[5, pl.ds(rbase, LANES)]
            for i in range(LANES):
                r = lbase + i
                sy = sy_vec[i]
                ts, us, vs = [], [], []
                du = jnp.zeros((LANES,), jnp.float32)
                dv = jnp.zeros((LANES,), jnp.float32)
                for j in range(DCH):
                    sl = pl.ds(j * LANES, LANES)
                    t = time_vmem[sy, sl]
                    rr = rl_rows[slot, r, sl]
                    u = ph_rows[slot, r, sl] + rr - pt_rows[slot, r, sl]
                    v = nh_rows[slot, r, sl] + rr - nt_rows[slot, r, sl]
                    du = du + u * t
                    dv = dv + v * t
                    ts.append(t)
                    us.append(u)
                    vs.append(v)
                du = _bcast_sum(du)
                dv = _bcast_sum(dv)
                pa = jnp.zeros((LANES,), jnp.float32)
                na = jnp.zeros((LANES,), jnp.float32)
                for j in range(DCH):
                    pa = pa + jnp.abs(us[j] - ts[j] * du)
                    na = na + jnp.abs(vs[j] - ts[j] * dv)
                pos_vec = jnp.where(lane == i, _bcast_sum(pa), pos_vec)
                neg_vec = jnp.where(lane == i, _bcast_sum(na), neg_vec)
            pos_buf[pl.ds(rbase, LANES)] = pos_vec
            neg_buf[pl.ds(rbase, LANES)] = neg_vec
            return c2

        lax.fori_loop(0, K // LANES, grp_body, 0)
        return carry

    lax.fori_loop(0, N_CHUNKS, chunk_body, 0)

    pltpu.sync_copy(pos_buf, pos_hbm.at[pl.ds(base, ROWS_PER_W)])
    pltpu.sync_copy(neg_buf, neg_hbm.at[pl.ds(base, ROWS_PER_W)])


@jax.jit
def kernel(ent_embeddings, rel_embeddings, time_embeddings,
           pos_head, pos_tail, rel, neg_head, neg_tail, start_year):
    mesh = plsc.VectorSubcoreMesh(core_axis_name="c", subcore_axis_name="s")
    fn = functools.partial(
        pl.kernel, mesh=mesh,
        out_type=(jax.ShapeDtypeStruct((B,), jnp.float32),
                  jax.ShapeDtypeStruct((B,), jnp.float32)),
        scratch_types=(
            [pltpu.VMEM((2, K, D), jnp.float32)] * 5
            + [pltpu.VMEM((T_ROWS, D), jnp.float32)]
            + [pltpu.VMEM((6, ROWS_PER_W), jnp.int32)]
            + [pltpu.VMEM_SHARED((500, D), jnp.float32)]
            + [pltpu.VMEM((ROWS_PER_W,), jnp.float32)] * 2
            + [pltpu.SemaphoreType.DMA] * 5
        ),
    )(_sc_kernel)
    idx_all = jnp.stack([pos_head[:, 0], pos_tail[:, 0], rel[:, 0],
                         neg_head[:, 0], neg_tail[:, 0], start_year])
    pos, neg = fn(ent_embeddings, rel_embeddings, time_embeddings, idx_all)
    return pos.reshape(B, 1), neg.reshape(B, 1)
